# baseline (speedup 1.0000x reference)
import functools

import jax
import jax.numpy as jnp
import numpy as np
from jax import lax
from jax.experimental import pallas as pl
from jax.experimental.pallas import tpu as pltpu
from jax.experimental.pallas import tpu_sc as plsc

N = 10000
E = 320000
D = 128
NS = 16
EPT = E // NS
CH = 112
NCH = EPT // CH
TE = EPT - NCH * CH
RPT = 624
TAIL = N - NS * RPT

_PERM = np.array([32 * g + (2 * o if o < 16 else 2 * (o - 16) + 1)
                  for g in range(D // 32) for o in range(32)], dtype=np.int32)

_mesh = plsc.VectorSubcoreMesh(core_axis_name="c", subcore_axis_name="s")

_GDN = lax.GatherDimensionNumbers(
    offset_dims=(), collapsed_slice_dims=(0,), start_index_map=(0,))


def _splat(vec, j):
    idx = jnp.full((16, 1), j, jnp.int32)
    return lax.gather(vec, idx, _GDN, (1,),
                      mode=lax.GatherScatterMode.PROMISE_IN_BOUNDS)


def _widen(u):
    a = lax.bitcast_convert_type(jnp.left_shift(u, 16), jnp.float32)
    b = lax.bitcast_convert_type(jnp.bitwise_and(u, jnp.int32(-65536)),
                                 jnp.float32)
    return a, b


@functools.partial(
    pl.kernel,
    out_type=(
        jax.ShapeDtypeStruct((N, D), jnp.float32),
        jax.ShapeDtypeStruct((N, D), jnp.float32),
    ),
    mesh=_mesh,
    compiler_params=pltpu.CompilerParams(use_tc_tiling_on_sc=False),
    scratch_types=[
        [pltpu.VMEM((CH,), jnp.int32) for _ in range(3)],
        [pltpu.VMEM((CH,), jnp.int32) for _ in range(3)],
        [pltpu.VMEM((CH,), jnp.float32) for _ in range(3)],
        [pltpu.VMEM((CH, D // 2), jnp.int32) for _ in range(2)],
        [pltpu.VMEM((CH, D), jnp.float32) for _ in range(2)],
        pltpu.VMEM((TE,), jnp.int32),
        pltpu.VMEM_SHARED((N, D), jnp.float32),
        [pltpu.SemaphoreType.DMA for _ in range(2)],
        [pltpu.SemaphoreType.DMA for _ in range(3)],
        [pltpu.SemaphoreType.DMA for _ in range(2)],
    ],
)
def _message_pass(gf_hbm, e_hbm, xh_hbm, mi_hbm, mo_hbm,
                  gidx, sidx, evr, rows_b, rows_f, sidx_t, acc,
                  gsem, isem, ssem):
    c = lax.axis_index("c")
    s = lax.axis_index("s")

    z = jnp.zeros((16,), jnp.float32)

    def _zrow(i, carry):
        for q in range(D // 16):
            rows_f[0][i, pl.ds(q * 16, 16)] = z
        return carry

    lax.fori_loop(0, CH, _zrow, 0)
    abase = s * RPT
    nfull = RPT // CH
    rem = RPT - nfull * CH
    for k in range(nfull):
        pltpu.sync_copy(rows_f[0], acc.at[pl.ds(abase + k * CH, CH)])
    pltpu.sync_copy(rows_f[0].at[pl.ds(0, rem)],
                    acc.at[pl.ds(abase + nfull * CH, rem)])

    @pl.when(s == NS - 1)
    def _():
        pltpu.sync_copy(rows_f[0].at[pl.ds(0, TAIL)], acc.at[pl.ds(NS * RPT, TAIL)])

    plsc.subcore_barrier()

    ebase = s * EPT
    goff = c * E + ebase
    soff = (1 - c) * E + ebase

    def _issue_idx(j, ix):
        b = j * CH
        pltpu.async_copy(gf_hbm.at[pl.ds(pl.multiple_of(goff + b, 8), CH)],
                         gidx[ix], isem[ix])
        pltpu.async_copy(gf_hbm.at[pl.ds(pl.multiple_of(soff + b, 8), CH)],
                         sidx[ix], isem[ix])
        pltpu.async_copy(e_hbm.at[pl.ds(pl.multiple_of(ebase + b, 8), CH)],
                         evr[ix], isem[ix])

    def _wait_idx(ix):
        pltpu.make_async_copy(gf_hbm.at[pl.ds(0, CH)], gidx[ix], isem[ix]).wait()
        pltpu.make_async_copy(gf_hbm.at[pl.ds(0, CH)], sidx[ix], isem[ix]).wait()
        pltpu.make_async_copy(e_hbm.at[pl.ds(0, CH)], evr[ix], isem[ix]).wait()

    def _issue_rows(ix, ib):
        pltpu.async_copy(xh_hbm.at[gidx[ix]], rows_b[ib], gsem[ib])

    def _wait_rows(ix, ib):
        pltpu.make_async_copy(xh_hbm.at[gidx[ix]], rows_b[ib], gsem[ib]).wait()

    def _scale(ib, jf, ix):
        bsrc = rows_b[ib]
        fdst = rows_f[jf]
        evb = evr[ix]

        def _grp(g16, carry):
            ev16 = evb[pl.ds(g16 * 16, 16)]
            for t in range(16):
                sc = _splat(ev16, t)
                r = g16 * 16 + t
                for g in range(D // 32):
                    u = bsrc[r, pl.ds(g * 16, 16)]
                    a, b = _widen(u)
                    fdst[r, pl.ds(2 * g * 16, 16)] = a * sc
                    fdst[r, pl.ds((2 * g + 1) * 16, 16)] = b * sc
            return carry

        lax.fori_loop(0, CH // 16, _grp, 0)

    def _scatter(jf, ix):
        pltpu.async_copy(rows_f[jf], acc.at[sidx[ix]], ssem[jf], add=True)

    def _wait_scatter(jf, ix):
        pltpu.make_async_copy(rows_f[jf], acc.at[sidx[ix]], ssem[jf]).wait()

    _issue_idx(0, 0)
    _issue_idx(1, 1)
    _wait_idx(0)
    _issue_rows(0, 0)

    def _group(g, carry):
        for i in range(6):
            j = g * 6 + i
            i2, i3 = i % 2, i % 3

            @pl.when(j < NCH)
            def _():
                @pl.when(j >= 1)
                def _():
                    _wait_scatter((i2 + 1) % 2, (i3 + 2) % 3)

                @pl.when(j + 2 < NCH)
                def _():
                    _issue_idx(j + 2, (i3 + 2) % 3)

                @pl.when(j + 1 < NCH)
                def _():
                    _wait_idx((i3 + 1) % 3)
                    _issue_rows((i3 + 1) % 3, (i2 + 1) % 2)

                _wait_rows(i3, i2)
                _scale(i2, i2, i3)
                _scatter(i2, i3)
        return carry

    lax.fori_loop(0, (NCH + 5) // 6, _group, 0)
    _wait_scatter((NCH - 1) % 2, (NCH - 1) % 3)

    tb = NCH * CH
    pltpu.sync_copy(gf_hbm.at[pl.ds(pl.multiple_of(goff + tb, 8), TE)],
                    gidx[0].at[pl.ds(0, TE)])
    pltpu.sync_copy(gf_hbm.at[pl.ds(pl.multiple_of(soff + tb, 8), TE)], sidx_t)
    pltpu.sync_copy(e_hbm.at[pl.ds(pl.multiple_of(ebase + tb, 8), TE)],
                    evr[0].at[pl.ds(0, TE)])
    pltpu.sync_copy(xh_hbm.at[gidx[0].at[pl.ds(0, TE)]], rows_b[0].at[pl.ds(0, TE)])

    def _tgrp(g16, carry):
        ev16 = evr[0][pl.ds(g16 * 16, 16)]
        for t in range(16):
            sc = _splat(ev16, t)
            r = g16 * 16 + t
            for g in range(D // 32):
                u = rows_b[0][r, pl.ds(g * 16, 16)]
                a, b = _widen(u)
                rows_f[0][r, pl.ds(2 * g * 16, 16)] = a * sc
                rows_f[0][r, pl.ds((2 * g + 1) * 16, 16)] = b * sc
        return carry

    lax.fori_loop(0, TE // 16, _tgrp, 0)
    pltpu.sync_copy(rows_f[0].at[pl.ds(0, TE)], acc.at[sidx_t], add=True)
    plsc.subcore_barrier()

    @pl.when(c == 0)
    def _():
        pltpu.sync_copy(acc.at[pl.ds(abase, RPT)], mi_hbm.at[pl.ds(abase, RPT)])

        @pl.when(s == NS - 1)
        def _():
            pltpu.sync_copy(acc.at[pl.ds(NS * RPT, TAIL)],
                            mi_hbm.at[pl.ds(NS * RPT, TAIL)])

    @pl.when(c == 1)
    def _():
        pltpu.sync_copy(acc.at[pl.ds(abase, RPT)], mo_hbm.at[pl.ds(abase, RPT)])

        @pl.when(s == NS - 1)
        def _():
            pltpu.sync_copy(acc.at[pl.ds(NS * RPT, TAIL)],
                            mo_hbm.at[pl.ds(NS * RPT, TAIL)])


_BR = 1000


def _mlp_body(mi, mo, x, W1, b1, W2, b2, W3, b3, W4, b4, out):
    ni = jnp.concatenate([mi[...], mo[...], x[...]], axis=1)
    h = jnp.tanh(jnp.dot(ni, W1[...], preferred_element_type=jnp.float32) + b1[...])
    h = jnp.tanh(jnp.dot(h, W2[...], preferred_element_type=jnp.float32) + b2[...])
    h = jnp.tanh(jnp.dot(h, W3[...], preferred_element_type=jnp.float32) + b3[...])
    h = jnp.tanh(jnp.dot(h, W4[...], preferred_element_type=jnp.float32) + b4[...])
    out[...] = h


def _mlp(mi, mo, x, W1, b1, W2, b2, W3, b3, W4, b4):
    row = pl.BlockSpec((_BR, D), lambda i: (i, 0))
    w1s = pl.BlockSpec((3 * D, D), lambda i: (0, 0))
    ws = pl.BlockSpec((D, D), lambda i: (0, 0))
    bs = pl.BlockSpec((1, D), lambda i: (0, 0))
    return pl.pallas_call(
        _mlp_body,
        grid=(N // _BR,),
        in_specs=[row, row, row, w1s, bs, ws, bs, ws, bs, ws, bs],
        out_specs=row,
        out_shape=jax.ShapeDtypeStruct((N, D), jnp.float32),
    )(mi, mo, x, W1, b1.reshape(1, D), W2, b2.reshape(1, D),
      W3, b3.reshape(1, D), W4, b4.reshape(1, D))


def kernel(x, e, edge_index, W1, b1, W2, b2, W3, b3, W4, b4):
    ei = edge_index.astype(jnp.int32)
    gf = ei.reshape(-1)
    xh = x.astype(jnp.bfloat16)
    xi = lax.bitcast_convert_type(xh.reshape(N, D // 2, 2), jnp.int32)
    mi, mo = _message_pass(gf, e, xi)
    perm = jnp.asarray(_PERM)
    W1p = jnp.concatenate([W1[:D][perm], W1[D:2 * D][perm], W1[2 * D:]], axis=0)
    return _mlp(mi, mo, x, W1p, b1, W2, b2, W3, b3, W4, b4)


# speedup vs baseline: 2.3638x; 2.3638x over previous
import functools

import jax
import jax.numpy as jnp
from jax import lax
from jax.experimental import pallas as pl
from jax.experimental.pallas import tpu as pltpu
from jax.experimental.pallas import tpu_sc as plsc

N = 10000
E = 320000
D = 128
NS = 16
EPT = E // NS
CH = 128
NCH = EPT // CH
TE = EPT - NCH * CH
RPT = 624
TAIL = N - NS * RPT
NB = 3

_mesh = plsc.VectorSubcoreMesh(core_axis_name="c", subcore_axis_name="s")

_GDN = lax.GatherDimensionNumbers(
    offset_dims=(), collapsed_slice_dims=(0,), start_index_map=(0,))


def _splat(vec, j):
    idx = jnp.full((16, 1), j, jnp.int32)
    return lax.gather(vec, idx, _GDN, (1,),
                      mode=lax.GatherScatterMode.PROMISE_IN_BOUNDS)


@functools.partial(
    pl.kernel,
    out_type=(
        jax.ShapeDtypeStruct((N, D), jnp.float32),
        jax.ShapeDtypeStruct((N, D), jnp.float32),
    ),
    mesh=_mesh,
    scratch_types=[
        [pltpu.VMEM((CH,), jnp.int32) for _ in range(NB)],
        [pltpu.VMEM((CH,), jnp.int32) for _ in range(NB)],
        [pltpu.VMEM((CH,), jnp.float32) for _ in range(NB)],
        [pltpu.VMEM((CH, D), jnp.float32) for _ in range(NB)],
        pltpu.VMEM((TE,), jnp.int32),
        pltpu.VMEM_SHARED((N, D), jnp.float32),
        [pltpu.SemaphoreType.DMA for _ in range(NB)],
        [pltpu.SemaphoreType.DMA for _ in range(NB)],
        [pltpu.SemaphoreType.DMA for _ in range(NB)],
    ],
)
def _message_pass(gf_hbm, e_hbm, x_hbm, mi_hbm, mo_hbm,
                  gidx, sidx, evr, rows, sidx_t, acc, gsem, isem, ssem):
    c = lax.axis_index("c")
    s = lax.axis_index("s")

    z = jnp.zeros((16,), jnp.float32)

    def _zrow(i, carry):
        for q in range(D // 16):
            rows[0][i, pl.ds(q * 16, 16)] = z
        return carry

    lax.fori_loop(0, CH, _zrow, 0)
    abase = s * RPT
    nfull = RPT // CH
    rem = RPT - nfull * CH
    for k in range(nfull):
        pltpu.sync_copy(rows[0], acc.at[pl.ds(abase + k * CH, CH)])
    pltpu.sync_copy(rows[0].at[pl.ds(0, rem)], acc.at[pl.ds(abase + nfull * CH, rem)])

    @pl.when(s == NS - 1)
    def _():
        pltpu.sync_copy(rows[0].at[pl.ds(0, TAIL)], acc.at[pl.ds(NS * RPT, TAIL)])

    plsc.subcore_barrier()

    ebase = s * EPT
    goff = c * E + ebase
    soff = (1 - c) * E + ebase

    def _issue_idx(j, bi):
        b = j * CH
        pltpu.async_copy(gf_hbm.at[pl.ds(pl.multiple_of(goff + b, 8), CH)],
                         gidx[bi], isem[bi])
        pltpu.async_copy(gf_hbm.at[pl.ds(pl.multiple_of(soff + b, 8), CH)],
                         sidx[bi], isem[bi])
        pltpu.async_copy(e_hbm.at[pl.ds(pl.multiple_of(ebase + b, 8), CH)],
                         evr[bi], isem[bi])

    def _wait_idx(bi):
        pltpu.make_async_copy(gf_hbm.at[pl.ds(0, CH)], gidx[bi], isem[bi]).wait()
        pltpu.make_async_copy(gf_hbm.at[pl.ds(0, CH)], sidx[bi], isem[bi]).wait()
        pltpu.make_async_copy(e_hbm.at[pl.ds(0, CH)], evr[bi], isem[bi]).wait()

    def _issue_rows(bi):
        pltpu.async_copy(x_hbm.at[gidx[bi]], rows[bi], gsem[bi])

    def _wait_rows(bi):
        pltpu.make_async_copy(x_hbm.at[gidx[bi]], rows[bi], gsem[bi]).wait()

    def _scale(bi):
        rbuf = rows[bi]
        evb = evr[bi]

        def _grp(g16, carry):
            ev16 = evb[pl.ds(g16 * 16, 16)]
            for t in range(16):
                sc = _splat(ev16, t)
                r = g16 * 16 + t
                for q in range(D // 16):
                    sl = pl.ds(q * 16, 16)
                    rbuf[r, sl] = rbuf[r, sl] * sc
            return carry

        lax.fori_loop(0, CH // 16, _grp, 0)

    def _scatter(bi):
        pltpu.async_copy(rows[bi], acc.at[sidx[bi]], ssem[bi], add=True)

    def _wait_scatter(bi):
        pltpu.make_async_copy(rows[bi], acc.at[sidx[bi]], ssem[bi]).wait()

    for bi in range(NB):
        _issue_idx(bi, bi)
    _wait_idx(0)
    _issue_rows(0)

    def _group(g, carry):
        for i in range(NB):
            j = g * NB + i
            ip = (i + 2) % NB
            inx = (i + 1) % NB

            @pl.when(j < NCH)
            def _():
                @pl.when(jnp.logical_and(j >= 1, j + 2 < NCH))
                def _():
                    _wait_scatter(ip)
                    _issue_idx(j + 2, ip)

                @pl.when(j + 1 < NCH)
                def _():
                    _wait_idx(inx)
                    _issue_rows(inx)

                _wait_rows(i)
                _scale(i)
                _scatter(i)
        return carry

    lax.fori_loop(0, (NCH + NB - 1) // NB, _group, 0)
    for bi in range(NB):
        _wait_scatter(bi)

    tb = NCH * CH
    pltpu.sync_copy(gf_hbm.at[pl.ds(pl.multiple_of(goff + tb, 8), TE)],
                    gidx[0].at[pl.ds(0, TE)])
    pltpu.sync_copy(gf_hbm.at[pl.ds(pl.multiple_of(soff + tb, 8), TE)], sidx_t)
    pltpu.sync_copy(e_hbm.at[pl.ds(pl.multiple_of(ebase + tb, 8), TE)],
                    evr[0].at[pl.ds(0, TE)])
    pltpu.sync_copy(x_hbm.at[gidx[0].at[pl.ds(0, TE)]], rows[0].at[pl.ds(0, TE)])

    def _tgrp(g16, carry):
        ev16 = evr[0][pl.ds(g16 * 16, 16)]
        for t in range(16):
            sc = _splat(ev16, t)
            r = g16 * 16 + t
            for q in range(D // 16):
                sl = pl.ds(q * 16, 16)
                rows[0][r, sl] = rows[0][r, sl] * sc
        return carry

    lax.fori_loop(0, TE // 16, _tgrp, 0)
    pltpu.sync_copy(rows[0].at[pl.ds(0, TE)], acc.at[sidx_t], add=True)
    plsc.subcore_barrier()

    @pl.when(c == 0)
    def _():
        pltpu.sync_copy(acc.at[pl.ds(abase, RPT)], mi_hbm.at[pl.ds(abase, RPT)])

        @pl.when(s == NS - 1)
        def _():
            pltpu.sync_copy(acc.at[pl.ds(NS * RPT, TAIL)],
                            mi_hbm.at[pl.ds(NS * RPT, TAIL)])

    @pl.when(c == 1)
    def _():
        pltpu.sync_copy(acc.at[pl.ds(abase, RPT)], mo_hbm.at[pl.ds(abase, RPT)])

        @pl.when(s == NS - 1)
        def _():
            pltpu.sync_copy(acc.at[pl.ds(NS * RPT, TAIL)],
                            mo_hbm.at[pl.ds(NS * RPT, TAIL)])


_BR = 1000


def _mlp_body(mi, mo, x, W1, b1, W2, b2, W3, b3, W4, b4, out):
    w1 = W1[...]
    h = jnp.tanh(jnp.dot(mi[...], w1[:D], preferred_element_type=jnp.float32)
                 + jnp.dot(mo[...], w1[D:2 * D], preferred_element_type=jnp.float32)
                 + jnp.dot(x[...], w1[2 * D:], preferred_element_type=jnp.float32)
                 + b1[...])
    h = jnp.tanh(jnp.dot(h, W2[...], preferred_element_type=jnp.float32) + b2[...])
    h = jnp.tanh(jnp.dot(h, W3[...], preferred_element_type=jnp.float32) + b3[...])
    h = jnp.tanh(jnp.dot(h, W4[...], preferred_element_type=jnp.float32) + b4[...])
    out[...] = h


def _mlp(mi, mo, x, W1, b1, W2, b2, W3, b3, W4, b4):
    row = pl.BlockSpec((_BR, D), lambda i: (i, 0))
    w1s = pl.BlockSpec((3 * D, D), lambda i: (0, 0))
    ws = pl.BlockSpec((D, D), lambda i: (0, 0))
    bs = pl.BlockSpec((1, D), lambda i: (0, 0))
    return pl.pallas_call(
        _mlp_body,
        grid=(N // _BR,),
        in_specs=[row, row, row, w1s, bs, ws, bs, ws, bs, ws, bs],
        out_specs=row,
        out_shape=jax.ShapeDtypeStruct((N, D), jnp.float32),
    )(mi, mo, x, W1, b1.reshape(1, D), W2, b2.reshape(1, D),
      W3, b3.reshape(1, D), W4, b4.reshape(1, D))


def kernel(x, e, edge_index, W1, b1, W2, b2, W3, b3, W4, b4):
    ei = edge_index.astype(jnp.int32)
    gf = ei.reshape(-1)
    mi, mo = _message_pass(gf, e, x)
    return _mlp(mi, mo, x, W1, b1, W2, b2, W3, b3, W4, b4)
